# baseline (speedup 1.0000x reference)
import functools

import jax
import jax.numpy as jnp
from jax import lax
from jax.experimental import pallas as pl
from jax.experimental.pallas import tpu as pltpu
from jax.experimental.pallas import tpu_sc as plsc

N_NODES = 50000
D = 64
DH = 32
E_EDGES = 800000
NC = 2
NS = 16
CHUNK = 128
NCHUNK = 400
EPT = NCHUNK * CHUNK
E_PAD = NS * EPT
SLAB = 8
NSLAB = NCHUNK // SLAB
NPAIR = NSLAB // 2
NBUF = 4
ZCHUNK = 200
NZ = N_NODES // ZCHUNK

_SC_MESH = plsc.VectorSubcoreMesh(core_axis_name="c", subcore_axis_name="s")


@functools.partial(
    pl.kernel,
    out_type=jax.ShapeDtypeStruct((NC, N_NODES, DH), jnp.float32),
    mesh=_SC_MESH,
    scratch_types=[
        pltpu.VMEM((SLAB, 3, CHUNK), jnp.int32),
        pltpu.VMEM((SLAB, 3, CHUNK), jnp.int32),
        pltpu.VMEM((NBUF, CHUNK, DH), jnp.float32),
        pltpu.VMEM((ZCHUNK, DH), jnp.float32),
        pltpu.VMEM_SHARED((N_NODES, DH), jnp.float32),
        pltpu.SemaphoreType.DMA,
        pltpu.SemaphoreType.DMA,
        pltpu.SemaphoreType.DMA,
        pltpu.SemaphoreType.DMA,
        pltpu.SemaphoreType.DMA,
        pltpu.SemaphoreType.DMA,
        pltpu.SemaphoreType.DMA,
        pltpu.SemaphoreType.DMA,
        pltpu.SemaphoreType.DMA,
    ],
    compiler_params=pltpu.CompilerParams(use_tc_tiling_on_sc=False,
                                         needs_layout_passes=False),
)
def _spmm_sc(ego_hbm, comb_hbm, out_hbm,
             idxa_v, idxb_v, rows_v, zbuf_v, acc,
             gsem0, gsem1, gsem2, gsem3, ssem0, ssem1, ssem2, ssem3,
             isem):
    cid = lax.axis_index("c")
    sid = lax.axis_index("s")
    coff = cid * N_NODES

    def _zfill(i, _):
        zbuf_v[i, pl.ds(0, 16)] = jnp.zeros((16,), jnp.float32)
        zbuf_v[i, pl.ds(16, 16)] = jnp.zeros((16,), jnp.float32)
        return 0
    lax.fori_loop(0, ZCHUNK, _zfill, 0)

    nk = (NZ - sid + NS - 1) // NS
    def _zcopy(k, _):
        idx = sid + k * NS
        pltpu.sync_copy(zbuf_v, acc.at[pl.ds(idx * ZCHUNK, ZCHUNK)])
        return 0
    lax.fori_loop(0, nk, _zcopy, 0)
    plsc.subcore_barrier()

    bufs = tuple(rows_v.at[b] for b in range(NBUF))
    gsems = (gsem0, gsem1, gsem2, gsem3)
    ssems = (ssem0, ssem1, ssem2, ssem3)
    cbase = sid * NCHUNK

    def _scale(rows_ref, idxb, k):
        def _scale16(t, _):
            vv = plsc.bitcast(idxb[k, 2, pl.ds(t * 16, 16)], jnp.float32)
            for i in range(16):
                e = t * 16 + i
                v = vv[i]
                r0 = rows_ref[e, pl.ds(0, 16)]
                rows_ref[e, pl.ds(0, 16)] = r0 * v
                r1 = rows_ref[e, pl.ds(16, 16)]
                rows_ref[e, pl.ds(16, 16)] = r1 * v
            return 0
        lax.fori_loop(0, CHUNK // 16, _scale16, 0)

    def _drain_scatter(par, idxb):
        pltpu.make_async_copy(bufs[par], acc.at[idxb.at[par, 1]],
                              ssems[par]).wait()

    def _drain_idx(idxb):
        pltpu.make_async_copy(comb_hbm.at[pl.ds(0, SLAB)], idxb,
                              isem).wait()

    def _do_slab(idxb, first_cond, prefetch_fn):
        for k in range(SLAB):
            for t in range(CHUNK // 16):
                cv = idxb[k, 0, pl.ds(t * 16, 16)]
                idxb[k, 0, pl.ds(t * 16, 16)] = cv + coff

        def _guarded_drain(b):
            if first_cond is None:
                _drain_scatter(b, idxb)
            else:
                @pl.when(first_cond)
                def _():
                    _drain_scatter(b, idxb)

        for k in range(NBUF - 1):
            _guarded_drain(k % NBUF)
            pltpu.async_copy(ego_hbm.at[idxb.at[k, 0]], bufs[k % NBUF],
                             gsems[k % NBUF])

        for k in range(SLAB):
            b = k % NBUF
            if k + NBUF - 1 < SLAB:
                nb = (k + NBUF - 1) % NBUF
                if k == 0:
                    _guarded_drain(nb)
                else:
                    _drain_scatter(nb, idxb)
                pltpu.async_copy(ego_hbm.at[idxb.at[k + NBUF - 1, 0]],
                                 bufs[nb], gsems[nb])
            if k == 2:
                prefetch_fn()
            pltpu.make_async_copy(ego_hbm.at[idxb.at[k, 0]],
                                  bufs[b], gsems[b]).wait()
            _scale(bufs[b], idxb, k)
            pltpu.async_copy(bufs[b], acc.at[idxb.at[k, 1]],
                             ssems[b], add=True)

    pltpu.sync_copy(comb_hbm.at[pl.ds(cbase, SLAB)], idxa_v)

    def _pair(p, _):
        def _prefetch_b():
            pltpu.async_copy(
                comb_hbm.at[pl.ds(cbase + (2 * p + 1) * SLAB, SLAB)],
                idxb_v, isem)

        def _prefetch_a():
            @pl.when(p + 1 < NPAIR)
            def _():
                pltpu.async_copy(
                    comb_hbm.at[pl.ds(cbase + (2 * p + 2) * SLAB, SLAB)],
                    idxa_v, isem)

        @pl.when(p > 0)
        def _():
            _drain_idx(idxa_v)
        _do_slab(idxa_v, p > 0, _prefetch_b)
        _drain_idx(idxb_v)
        _do_slab(idxb_v, None, _prefetch_a)
        return 0
    lax.fori_loop(0, NPAIR, _pair, 0)

    for b in range(NBUF):
        _drain_scatter(b, idxb_v)
    plsc.subcore_barrier()

    def _wcopy(k, _):
        idx = sid + k * NS
        pltpu.sync_copy(acc.at[pl.ds(idx * ZCHUNK, ZCHUNK)],
                        out_hbm.at[cid, pl.ds(idx * ZCHUNK, ZCHUNK)])
        return 0
    lax.fori_loop(0, nk, _wcopy, 0)


_BN = 2000
_GRID = N_NODES // _BN


def _layer_body(e_ref, eps_ref, w1_ref, b1_ref, w2_ref, b2_ref,
                p_ref, sum_ref):
    x = jnp.concatenate([e_ref[0], e_ref[1]], axis=1)
    h = jnp.maximum(jnp.dot(x, w1_ref[...],
                            preferred_element_type=jnp.float32)
                    + b1_ref[...], 0.0)
    logits = jnp.dot(h, w2_ref[...],
                     preferred_element_type=jnp.float32) + b2_ref[...]
    eps = eps_ref[0]
    gate = eps / (eps + (1.0 - eps) * jnp.exp(-logits))
    p = x * gate
    p_ref[0] = p[:, :DH]
    p_ref[1] = p[:, DH:]

    @pl.when(pl.program_id(0) == 0)
    def _():
        sum_ref[0, 0] = 0.0
    sum_ref[0, 0] += jnp.sum(logits)


_layer1_tc = pl.pallas_call(
    _layer_body,
    grid=(_GRID,),
    in_specs=[
        pl.BlockSpec((NC, _BN, DH), lambda i: (0, i, 0)),
        pl.BlockSpec((1, _BN, D), lambda i: (0, i, 0)),
        pl.BlockSpec((D, D), lambda i: (0, 0)),
        pl.BlockSpec((1, D), lambda i: (0, 0)),
        pl.BlockSpec((D, D), lambda i: (0, 0)),
        pl.BlockSpec((1, D), lambda i: (0, 0)),
    ],
    out_specs=[
        pl.BlockSpec((NC, _BN, DH), lambda i: (0, i, 0)),
        pl.BlockSpec((1, 1), lambda i: (0, 0),
                     memory_space=pltpu.SMEM),
    ],
    out_shape=[
        jax.ShapeDtypeStruct((NC, N_NODES, DH), jnp.float32),
        jax.ShapeDtypeStruct((1, 1), jnp.float32),
    ],
)


U_NODES = 25000
_BN2 = 1000
_GRID2 = U_NODES // _BN2
_HOFF = U_NODES // _BN2


def _final_body(ego0_u, ego0_i, e1_u, e1_i, e2_u, e2_i, p1_u, p1_i,
                eps_u, eps_i, w1_ref, b1_ref, w2_ref, b2_ref,
                ua_ref, ia_ref, up_ref, ip_ref, sum_ref):
    def _half(ego0_ref, e1_ref, e2_ref, p1_ref, eps_ref,
              emb_ref, pert_ref):
        x = jnp.concatenate([e2_ref[0], e2_ref[1]], axis=1)
        h = jnp.maximum(jnp.dot(x, w1_ref[...],
                                preferred_element_type=jnp.float32)
                        + b1_ref[...], 0.0)
        logits = jnp.dot(h, w2_ref[...],
                         preferred_element_type=jnp.float32) + b2_ref[...]
        eps = eps_ref[0]
        gate = eps / (eps + (1.0 - eps) * jnp.exp(-logits))
        p2 = x * gate
        e1 = jnp.concatenate([e1_ref[0], e1_ref[1]], axis=1)
        p1 = jnp.concatenate([p1_ref[0], p1_ref[1]], axis=1)
        emb_ref[...] = (ego0_ref[...] + e1 + x) * (1.0 / 3.0)
        pert_ref[...] = (p1 + p2) * 0.5
        return jnp.sum(logits)

    s = _half(ego0_u, e1_u, e2_u, p1_u, eps_u, ua_ref, up_ref)
    s += _half(ego0_i, e1_i, e2_i, p1_i, eps_i, ia_ref, ip_ref)

    @pl.when(pl.program_id(0) == 0)
    def _():
        sum_ref[0, 0] = 0.0
    sum_ref[0, 0] += s


_final_tc = pl.pallas_call(
    _final_body,
    grid=(_GRID2,),
    in_specs=[
        pl.BlockSpec((_BN2, D), lambda i: (i, 0)),
        pl.BlockSpec((_BN2, D), lambda i: (i + _HOFF, 0)),
        pl.BlockSpec((NC, _BN2, DH), lambda i: (0, i, 0)),
        pl.BlockSpec((NC, _BN2, DH), lambda i: (0, i + _HOFF, 0)),
        pl.BlockSpec((NC, _BN2, DH), lambda i: (0, i, 0)),
        pl.BlockSpec((NC, _BN2, DH), lambda i: (0, i + _HOFF, 0)),
        pl.BlockSpec((NC, _BN2, DH), lambda i: (0, i, 0)),
        pl.BlockSpec((NC, _BN2, DH), lambda i: (0, i + _HOFF, 0)),
        pl.BlockSpec((1, _BN2, D), lambda i: (1, i, 0)),
        pl.BlockSpec((1, _BN2, D), lambda i: (1, i + _HOFF, 0)),
        pl.BlockSpec((D, D), lambda i: (0, 0)),
        pl.BlockSpec((1, D), lambda i: (0, 0)),
        pl.BlockSpec((D, D), lambda i: (0, 0)),
        pl.BlockSpec((1, D), lambda i: (0, 0)),
    ],
    out_specs=[
        pl.BlockSpec((_BN2, D), lambda i: (i, 0)),
        pl.BlockSpec((_BN2, D), lambda i: (i, 0)),
        pl.BlockSpec((_BN2, D), lambda i: (i, 0)),
        pl.BlockSpec((_BN2, D), lambda i: (i, 0)),
        pl.BlockSpec((1, 1), lambda i: (0, 0),
                     memory_space=pltpu.SMEM),
    ],
    out_shape=[
        jax.ShapeDtypeStruct((U_NODES, D), jnp.float32),
        jax.ShapeDtypeStruct((U_NODES, D), jnp.float32),
        jax.ShapeDtypeStruct((U_NODES, D), jnp.float32),
        jax.ShapeDtypeStruct((U_NODES, D), jnp.float32),
        jax.ShapeDtypeStruct((1, 1), jnp.float32),
    ],
)


@jax.jit
def kernel(user_emb, item_emb, W1, b1, W2, b2, edge_vals, eps, edge_index):
    n_user = user_emb.shape[0]
    row = edge_index[0].astype(jnp.int32)
    col = edge_index[1].astype(jnp.int32)
    pad = E_PAD - E_EDGES
    ipad = jnp.zeros((pad,), jnp.int32)
    rowp = jnp.concatenate([row, ipad]).reshape(E_PAD // CHUNK, CHUNK)
    colp = jnp.concatenate([col, ipad]).reshape(E_PAD // CHUNK, CHUNK)
    vbits = lax.bitcast_convert_type(
        jnp.concatenate([edge_vals, jnp.zeros((pad,), jnp.float32)]),
        jnp.int32).reshape(E_PAD // CHUNK, CHUNK)
    comb = jnp.stack([colp, rowp, vbits], axis=1)

    ego0 = jnp.concatenate([user_emb, item_emb], axis=0)
    ego0_stack = jnp.concatenate([ego0[:, :DH], ego0[:, DH:]], axis=0)

    e1 = _spmm_sc(ego0_stack, comb)
    p1, s0 = _layer1_tc(e1, eps, W1[0], b1[0][None, :], W2[0],
                        b2[0][None, :])
    e2 = _spmm_sc(p1.reshape(NC * N_NODES, DH), comb)
    user_all, item_all, user_pert, item_pert, s1 = _final_tc(
        ego0, ego0, e1, e1, e2, e2, p1, p1, eps, eps,
        W1[1], b1[1][None, :], W2[1], b2[1][None, :])

    del n_user
    mask_mean = (s0[0, 0] + s1[0, 0]) / jnp.float32(N_NODES * D)
    return (user_all, item_all, user_pert, item_pert, mask_mean)


# speedup vs baseline: 1.2263x; 1.2263x over previous
import functools

import jax
import jax.numpy as jnp
from jax import lax
from jax.experimental import pallas as pl
from jax.experimental.pallas import tpu as pltpu
from jax.experimental.pallas import tpu_sc as plsc

N_NODES = 50000
D = 64
DH = 32
E_EDGES = 800000
NC = 2
NS = 16
CHUNK = 128
NCHUNK = 392
EPT = NCHUNK * CHUNK
E_PAD = NS * EPT
SLAB = 14
NSLAB = NCHUNK // SLAB
NPAIR = NSLAB // 2
ZCHUNK = 200
NZ = N_NODES // ZCHUNK

_SC_MESH = plsc.VectorSubcoreMesh(core_axis_name="c", subcore_axis_name="s")


@functools.partial(
    pl.kernel,
    out_type=jax.ShapeDtypeStruct((NC, N_NODES, DH), jnp.float32),
    mesh=_SC_MESH,
    scratch_types=[
        pltpu.VMEM((SLAB, 3, CHUNK), jnp.int32),
        pltpu.VMEM((SLAB, 3, CHUNK), jnp.int32),
        pltpu.VMEM((CHUNK, DH), jnp.float32),
        pltpu.VMEM((CHUNK, DH), jnp.float32),
        pltpu.VMEM((ZCHUNK, DH), jnp.float32),
        pltpu.VMEM_SHARED((N_NODES, DH), jnp.float32),
        pltpu.SemaphoreType.DMA,
        pltpu.SemaphoreType.DMA,
        pltpu.SemaphoreType.DMA,
        pltpu.SemaphoreType.DMA,
        pltpu.SemaphoreType.DMA,
    ],
    compiler_params=pltpu.CompilerParams(use_tc_tiling_on_sc=False,
                                         needs_layout_passes=False),
)
def _spmm_sc(ego_hbm, comb_hbm, out_hbm,
             idxa_v, idxb_v, rows0_v, rows1_v, zbuf_v, acc,
             gsem0, gsem1, ssem0, ssem1, isem):
    cid = lax.axis_index("c")
    sid = lax.axis_index("s")
    coff = cid * N_NODES

    def _zfill(i, _):
        zbuf_v[i, pl.ds(0, 16)] = jnp.zeros((16,), jnp.float32)
        zbuf_v[i, pl.ds(16, 16)] = jnp.zeros((16,), jnp.float32)
        return 0
    lax.fori_loop(0, ZCHUNK, _zfill, 0)

    nk = (NZ - sid + NS - 1) // NS
    def _zcopy(k, _):
        idx = sid + k * NS
        pltpu.sync_copy(zbuf_v, acc.at[pl.ds(idx * ZCHUNK, ZCHUNK)])
        return 0
    lax.fori_loop(0, nk, _zcopy, 0)
    plsc.subcore_barrier()

    bufs = (rows0_v, rows1_v)
    gsems = (gsem0, gsem1)
    ssems = (ssem0, ssem1)
    cbase = sid * NCHUNK

    def _scale(rows_ref, idxb, k):
        def _scale16(t, _):
            vv = plsc.bitcast(idxb[k, 2, pl.ds(t * 16, 16)], jnp.float32)
            for i in range(16):
                e = t * 16 + i
                v = vv[i]
                r0 = rows_ref[e, pl.ds(0, 16)]
                rows_ref[e, pl.ds(0, 16)] = r0 * v
                r1 = rows_ref[e, pl.ds(16, 16)]
                rows_ref[e, pl.ds(16, 16)] = r1 * v
            return 0
        lax.fori_loop(0, CHUNK // 16, _scale16, 0)

    def _drain_scatter(par, idxb):
        pltpu.make_async_copy(bufs[par], acc.at[idxb.at[par, 1]],
                              ssems[par]).wait()

    def _drain_idx(idxb):
        pltpu.make_async_copy(comb_hbm.at[pl.ds(0, SLAB)], idxb,
                              isem).wait()

    def _do_slab(idxb, s, first_cond, prefetch_fn):
        for k in range(SLAB):
            for t in range(CHUNK // 16):
                cv = idxb[k, 0, pl.ds(t * 16, 16)]
                idxb[k, 0, pl.ds(t * 16, 16)] = cv + coff

        if first_cond is None:
            _drain_scatter(0, idxb)
        else:
            @pl.when(first_cond)
            def _():
                _drain_scatter(0, idxb)
        pltpu.async_copy(ego_hbm.at[idxb.at[0, 0]], bufs[0], gsems[0])

        for k in range(SLAB):
            par = k % 2
            npar = 1 - par
            if k + 1 < SLAB:
                if k == 0 and first_cond is not None:
                    @pl.when(first_cond)
                    def _():
                        _drain_scatter(1, idxb)
                else:
                    _drain_scatter(npar, idxb)
                pltpu.async_copy(ego_hbm.at[idxb.at[k + 1, 0]],
                                 bufs[npar], gsems[npar])
            if k == 2:
                prefetch_fn()
            pltpu.make_async_copy(ego_hbm.at[idxb.at[k, 0]],
                                  bufs[par], gsems[par]).wait()
            _scale(bufs[par], idxb, k)
            pltpu.async_copy(bufs[par], acc.at[idxb.at[k, 1]],
                             ssems[par], add=True)

    pltpu.sync_copy(comb_hbm.at[pl.ds(cbase, SLAB)], idxa_v)

    def _pair(p, _):
        def _prefetch_b():
            pltpu.async_copy(
                comb_hbm.at[pl.ds(cbase + (2 * p + 1) * SLAB, SLAB)],
                idxb_v, isem)

        def _prefetch_a():
            @pl.when(p + 1 < NPAIR)
            def _():
                pltpu.async_copy(
                    comb_hbm.at[pl.ds(cbase + (2 * p + 2) * SLAB, SLAB)],
                    idxa_v, isem)

        @pl.when(p > 0)
        def _():
            _drain_idx(idxa_v)
        _do_slab(idxa_v, 2 * p, p > 0, _prefetch_b)
        _drain_idx(idxb_v)
        _do_slab(idxb_v, 2 * p + 1, None, _prefetch_a)
        return 0
    lax.fori_loop(0, NPAIR, _pair, 0)

    _drain_scatter(0, idxb_v)
    _drain_scatter(1, idxb_v)
    plsc.subcore_barrier()

    def _wcopy(k, _):
        idx = sid + k * NS
        pltpu.sync_copy(acc.at[pl.ds(idx * ZCHUNK, ZCHUNK)],
                        out_hbm.at[cid, pl.ds(idx * ZCHUNK, ZCHUNK)])
        return 0
    lax.fori_loop(0, nk, _wcopy, 0)


_BN = 2000
_GRID = N_NODES // _BN


def _layer_body(e_ref, eps_ref, w1_ref, b1_ref, w2_ref, b2_ref,
                p_ref, sum_ref):
    x = jnp.concatenate([e_ref[0], e_ref[1]], axis=1)
    h = jnp.maximum(jnp.dot(x, w1_ref[...],
                            preferred_element_type=jnp.float32)
                    + b1_ref[...], 0.0)
    logits = jnp.dot(h, w2_ref[...],
                     preferred_element_type=jnp.float32) + b2_ref[...]
    eps = eps_ref[0]
    gate = eps / (eps + (1.0 - eps) * jnp.exp(-logits))
    p = x * gate
    p_ref[0] = p[:, :DH]
    p_ref[1] = p[:, DH:]

    @pl.when(pl.program_id(0) == 0)
    def _():
        sum_ref[0, 0] = 0.0
    sum_ref[0, 0] += jnp.sum(logits)


_layer1_tc = pl.pallas_call(
    _layer_body,
    grid=(_GRID,),
    in_specs=[
        pl.BlockSpec((NC, _BN, DH), lambda i: (0, i, 0)),
        pl.BlockSpec((1, _BN, D), lambda i: (0, i, 0)),
        pl.BlockSpec((D, D), lambda i: (0, 0)),
        pl.BlockSpec((1, D), lambda i: (0, 0)),
        pl.BlockSpec((D, D), lambda i: (0, 0)),
        pl.BlockSpec((1, D), lambda i: (0, 0)),
    ],
    out_specs=[
        pl.BlockSpec((NC, _BN, DH), lambda i: (0, i, 0)),
        pl.BlockSpec((1, 1), lambda i: (0, 0),
                     memory_space=pltpu.SMEM),
    ],
    out_shape=[
        jax.ShapeDtypeStruct((NC, N_NODES, DH), jnp.float32),
        jax.ShapeDtypeStruct((1, 1), jnp.float32),
    ],
)


U_NODES = 25000
_BN2 = 1000
_GRID2 = U_NODES // _BN2
_HOFF = U_NODES // _BN2


def _final_body(ego0_u, ego0_i, e1_u, e1_i, e2_u, e2_i, p1_u, p1_i,
                eps_u, eps_i, w1_ref, b1_ref, w2_ref, b2_ref,
                ua_ref, ia_ref, up_ref, ip_ref, sum_ref):
    def _half(ego0_ref, e1_ref, e2_ref, p1_ref, eps_ref,
              emb_ref, pert_ref):
        x = jnp.concatenate([e2_ref[0], e2_ref[1]], axis=1)
        h = jnp.maximum(jnp.dot(x, w1_ref[...],
                                preferred_element_type=jnp.float32)
                        + b1_ref[...], 0.0)
        logits = jnp.dot(h, w2_ref[...],
                         preferred_element_type=jnp.float32) + b2_ref[...]
        eps = eps_ref[0]
        gate = eps / (eps + (1.0 - eps) * jnp.exp(-logits))
        p2 = x * gate
        e1 = jnp.concatenate([e1_ref[0], e1_ref[1]], axis=1)
        p1 = jnp.concatenate([p1_ref[0], p1_ref[1]], axis=1)
        emb_ref[...] = (ego0_ref[...] + e1 + x) * (1.0 / 3.0)
        pert_ref[...] = (p1 + p2) * 0.5
        return jnp.sum(logits)

    s = _half(ego0_u, e1_u, e2_u, p1_u, eps_u, ua_ref, up_ref)
    s += _half(ego0_i, e1_i, e2_i, p1_i, eps_i, ia_ref, ip_ref)

    @pl.when(pl.program_id(0) == 0)
    def _():
        sum_ref[0, 0] = 0.0
    sum_ref[0, 0] += s


_final_tc = pl.pallas_call(
    _final_body,
    grid=(_GRID2,),
    in_specs=[
        pl.BlockSpec((_BN2, D), lambda i: (i, 0)),
        pl.BlockSpec((_BN2, D), lambda i: (i + _HOFF, 0)),
        pl.BlockSpec((NC, _BN2, DH), lambda i: (0, i, 0)),
        pl.BlockSpec((NC, _BN2, DH), lambda i: (0, i + _HOFF, 0)),
        pl.BlockSpec((NC, _BN2, DH), lambda i: (0, i, 0)),
        pl.BlockSpec((NC, _BN2, DH), lambda i: (0, i + _HOFF, 0)),
        pl.BlockSpec((NC, _BN2, DH), lambda i: (0, i, 0)),
        pl.BlockSpec((NC, _BN2, DH), lambda i: (0, i + _HOFF, 0)),
        pl.BlockSpec((1, _BN2, D), lambda i: (1, i, 0)),
        pl.BlockSpec((1, _BN2, D), lambda i: (1, i + _HOFF, 0)),
        pl.BlockSpec((D, D), lambda i: (0, 0)),
        pl.BlockSpec((1, D), lambda i: (0, 0)),
        pl.BlockSpec((D, D), lambda i: (0, 0)),
        pl.BlockSpec((1, D), lambda i: (0, 0)),
    ],
    out_specs=[
        pl.BlockSpec((_BN2, D), lambda i: (i, 0)),
        pl.BlockSpec((_BN2, D), lambda i: (i, 0)),
        pl.BlockSpec((_BN2, D), lambda i: (i, 0)),
        pl.BlockSpec((_BN2, D), lambda i: (i, 0)),
        pl.BlockSpec((1, 1), lambda i: (0, 0),
                     memory_space=pltpu.SMEM),
    ],
    out_shape=[
        jax.ShapeDtypeStruct((U_NODES, D), jnp.float32),
        jax.ShapeDtypeStruct((U_NODES, D), jnp.float32),
        jax.ShapeDtypeStruct((U_NODES, D), jnp.float32),
        jax.ShapeDtypeStruct((U_NODES, D), jnp.float32),
        jax.ShapeDtypeStruct((1, 1), jnp.float32),
    ],
)


@jax.jit
def kernel(user_emb, item_emb, W1, b1, W2, b2, edge_vals, eps, edge_index):
    n_user = user_emb.shape[0]
    row = edge_index[0].astype(jnp.int32)
    col = edge_index[1].astype(jnp.int32)
    pad = E_PAD - E_EDGES
    ipad = jnp.zeros((pad,), jnp.int32)
    rowp = jnp.concatenate([row, ipad]).reshape(E_PAD // CHUNK, CHUNK)
    colp = jnp.concatenate([col, ipad]).reshape(E_PAD // CHUNK, CHUNK)
    vbits = lax.bitcast_convert_type(
        jnp.concatenate([edge_vals, jnp.zeros((pad,), jnp.float32)]),
        jnp.int32).reshape(E_PAD // CHUNK, CHUNK)
    comb = jnp.stack([colp, rowp, vbits], axis=1)

    ego0 = jnp.concatenate([user_emb, item_emb], axis=0)
    ego0_stack = jnp.concatenate([ego0[:, :DH], ego0[:, DH:]], axis=0)

    e1 = _spmm_sc(ego0_stack, comb)
    p1, s0 = _layer1_tc(e1, eps, W1[0], b1[0][None, :], W2[0],
                        b2[0][None, :])
    e2 = _spmm_sc(p1.reshape(NC * N_NODES, DH), comb)
    user_all, item_all, user_pert, item_pert, s1 = _final_tc(
        ego0, ego0, e1, e1, e2, e2, p1, p1, eps, eps,
        W1[1], b1[1][None, :], W2[1], b2[1][None, :])

    del n_user
    mask_mean = (s0[0, 0] + s1[0, 0]) / jnp.float32(N_NODES * D)
    return (user_all, item_all, user_pert, item_pert, mask_mean)
